# baseline (speedup 1.0000x reference)
import functools

import jax
import jax.numpy as jnp
from jax import lax
from jax.experimental import pallas as pl
from jax.experimental.pallas import tpu as pltpu
from jax.experimental.pallas import tpu_sc as plsc

N = 10000
E = 320000
D = 128

NC = 2
NS = 16
NW = NC * NS
EW = E // NW
C = 80
NCH = EW // C
NP = 10240
RPT = NP // NS
NB = 3
NE = 6
LANES = 16


def _sc_scatter_body(with_weight, y_hbm, ed_hbm, w_hbm, out_hbm,
                     ed_v, w_v, rows_0, rows_1, rows_2,
                     acc_sh,
                     es_0, es_1, es_2, es_3, es_4, es_5,
                     ws_0, ws_1, ws_2, ws_3, ws_4, ws_5,
                     gs_0, gs_1, gs_2, ss_0, ss_1, ss_2):
    cid = lax.axis_index("c")
    sid = lax.axis_index("s")
    wid = sid * NC + cid
    rows = (rows_0, rows_1, rows_2)
    esem = (es_0, es_1, es_2, es_3, es_4, es_5)
    wsem = (ws_0, ws_1, ws_2, ws_3, ws_4, ws_5)
    gsem = (gs_0, gs_1, gs_2)
    ssem = (ss_0, ss_1, ss_2)

    def start_edata(j, be):
        pltpu.async_copy(ed_hbm.at[wid, j], ed_v.at[be], esem[be])
        if with_weight:
            pltpu.async_copy(w_hbm.at[wid, j], w_v.at[be], wsem[be])

    def wait_edata(j, be):
        pltpu.make_async_copy(ed_hbm.at[wid, j], ed_v.at[be],
                              esem[be]).wait()
        if with_weight:
            pltpu.make_async_copy(w_hbm.at[wid, j], w_v.at[be],
                                  wsem[be]).wait()

    def start_gather(j, b, be):
        pltpu.async_copy(y_hbm.at[ed_v.at[be, 0, 0]], rows[b], gsem[b])

    def wait_gather(j, b, be):
        pltpu.make_async_copy(y_hbm.at[ed_v.at[be, 0, 0]], rows[b],
                              gsem[b]).wait()

    def scale(b, be):
        r = rows[b]

        def group_body(g, _):
            wv = w_v[be, 0, pl.ds(g * LANES, LANES)]
            for el in range(LANES):
                e = g * LANES + el
                wb = jnp.full((LANES,), wv[el], jnp.float32)
                for k in range(D // LANES):
                    sl = pl.ds(k * LANES, LANES)
                    r[e, sl] = r[e, sl] * wb
            return 0
        lax.fori_loop(0, C // LANES, group_body, 0)

    def start_scatter(j, b, be):
        pltpu.async_copy(rows[b], acc_sh.at[ed_v.at[be, 1, 0]], ssem[b],
                         add=True)

    def wait_scatter(j, b, be):
        pltpu.make_async_copy(rows[b], acc_sh.at[ed_v.at[be, 1, 0]],
                              ssem[b]).wait()

    def lane(j, b, be, first=False, do_g=True, do_e=True):
        if do_g:
            wait_edata(j + 2, (be + 2) % NE)
        wait_gather(j, b, be)
        if not first:
            wait_scatter(j - 1, (b + 2) % NB, (be + 5) % NE)
        if do_g:
            start_gather(j + 2, (b + 2) % NB, (be + 2) % NE)
        if do_e:
            start_edata(j + 4, (be + 4) % NE)
        if with_weight:
            scale(b, be)
        start_scatter(j, b, be)

    for k in range(4):
        start_edata(k, k)

    def zero_body(i, _):
        rows_2[i // 8, pl.ds((i % 8) * LANES, LANES)] = jnp.zeros(
            (LANES,), jnp.float32)
        return 0
    lax.fori_loop(0, C * 8, zero_body, 0)
    for z in range(RPT // C):
        pltpu.async_copy(rows_2, acc_sh.at[pl.ds(sid * RPT + z * C, C)],
                         ss_0)
    for k in range(2):
        wait_edata(k, k)
        start_gather(k, k, k)
    for z in range(RPT // C):
        pltpu.make_async_copy(rows_2, acc_sh.at[pl.ds(sid * RPT + z * C, C)],
                              ss_0).wait()
    plsc.subcore_barrier()

    lane(0, 0, 0, first=True)
    lane(1, 1, 1)

    RS = 2
    RR = (NCH - RS - 4) // NE

    def ring_body(t, _):
        j0 = RS + NE * t
        for l in range(NE):
            lane(j0 + l, (RS + l) % NB, (RS + l) % NE)
        return 0
    lax.fori_loop(0, RR, ring_body, 0)

    for j in range(RS + NE * RR, NCH):
        lane(j, j % NB, j % NE, do_g=(j + 2 < NCH), do_e=(j + 4 < NCH))
    wait_scatter(NCH - 1, (NCH - 1) % NB, (NCH - 1) % NE)
    plsc.subcore_barrier()

    def rd(z, b):
        return (acc_sh.at[pl.ds(sid * RPT + z * C, C)], rows[b], gsem[b])

    def wr(z, b):
        return (rows[b], out_hbm.at[cid, pl.ds(sid * RPT + z * C, C)],
                ssem[b])

    pltpu.async_copy(*rd(0, 0))
    for z in range(RPT // C):
        b = z % 2
        pltpu.make_async_copy(*rd(z, b)).wait()
        if z > 0:
            pltpu.make_async_copy(*wr(z - 1, 1 - b)).wait()
        pltpu.async_copy(*wr(z, b))
        if z + 1 < RPT // C:
            pltpu.async_copy(*rd(z + 1, 1 - b))
    pltpu.make_async_copy(*wr(RPT // C - 1, (RPT // C - 1) % 2)).wait()


def _make_sc_scatter(with_weight):
    mesh = plsc.VectorSubcoreMesh(core_axis_name="c", subcore_axis_name="s",
                                  num_cores=NC, num_subcores=NS)
    scratch = (
        [pltpu.VMEM((NE, 2, 1, C), jnp.int32)]
        + [pltpu.VMEM((NE, 1, C), jnp.float32)]
        + [pltpu.VMEM((C, D), jnp.float32) for _ in range(NB)]
        + [pltpu.VMEM_SHARED((NP, D), jnp.float32)]
        + [pltpu.SemaphoreType.DMA] * (2 * NE + 2 * NB)
    )
    return pl.kernel(
        functools.partial(_sc_scatter_body, with_weight),
        out_type=jax.ShapeDtypeStruct((NC, NP, D), jnp.float32),
        mesh=mesh,
        scratch_types=scratch,
        name="sc_scatter_w" if with_weight else "sc_scatter",
    )


_sc_scatter_weighted = _make_sc_scatter(True)
_sc_scatter_plain = _make_sc_scatter(False)


def _mm(a, w):
    return lax.dot_general(a, w, (((1,), (1,)), ((), ())),
                           preferred_element_type=jnp.float32)


def _tc_root(x_ref, wq_ref, br_ref, bq_ref, r_ref):
    r_ref[...] = _mm(x_ref[...], wq_ref[...]) + br_ref[...] + bq_ref[...]


def _tc_merge(p_ref, r_ref, wr_ref, h_ref):
    agg = p_ref[0, :N, :] + p_ref[1, :N, :]
    h_ref[...] = jnp.maximum(_mm(agg, wr_ref[...]) + r_ref[...], 0.0)


_nd = jax.ShapeDtypeStruct((N, D), jnp.float32)
_tc_root_call = pl.pallas_call(_tc_root, out_shape=_nd)
_tc_merge_call = pl.pallas_call(_tc_merge, out_shape=_nd)


def kernel(x, edge_index, edge_weight, W1_rel, b1_rel, W1_root, b1_root,
           W2_rel, b2_rel, W2_root, b2_root):
    src = edge_index[0].reshape(NW, NCH, 1, 1, C)
    dst = edge_index[1].reshape(NW, NCH, 1, 1, C)
    ed = jnp.concatenate([src, dst], axis=2)
    w = edge_weight.reshape(NW, NCH, 1, C)

    p1 = _sc_scatter_weighted(x, ed, w)
    r1 = _tc_root_call(x, W1_root, b1_rel.reshape(1, D), b1_root.reshape(1, D))
    h1 = _tc_merge_call(p1, r1, W1_rel)
    p2 = _sc_scatter_plain(h1, ed, w)
    r2 = _tc_root_call(h1, W2_root, b2_rel.reshape(1, D),
                       b2_root.reshape(1, D))
    return _tc_merge_call(p2, r2, W2_rel)


# speedup vs baseline: 1.0349x; 1.0349x over previous
import functools

import jax
import jax.numpy as jnp
from jax import lax
from jax.experimental import pallas as pl
from jax.experimental.pallas import tpu as pltpu
from jax.experimental.pallas import tpu_sc as plsc

N = 10000
E = 320000
D = 128

NC = 2
NS = 16
NW = NC * NS
EW = E // NW
C = 80
NCH = EW // C
NP = 10240
RPT = NP // NS
NB = 3
NE = 6
LANES = 16


def _sc_scatter_body(with_weight, y_hbm, src_hbm, dst_hbm, w_hbm, out_hbm,
                     src_v, dst_v, w_v, rows_0, rows_1, rows_2,
                     acc_sh,
                     es_0, es_1, es_2, es_3, es_4, es_5,
                     ds_0, ds_1, ds_2, ds_3, ds_4, ds_5,
                     ws_0, ws_1, ws_2, ws_3, ws_4, ws_5,
                     gs_0, gs_1, gs_2, ss_0, ss_1, ss_2):
    cid = lax.axis_index("c")
    sid = lax.axis_index("s")
    wid = sid * NC + cid
    rows = (rows_0, rows_1, rows_2)
    esem = (es_0, es_1, es_2, es_3, es_4, es_5)
    dsem = (ds_0, ds_1, ds_2, ds_3, ds_4, ds_5)
    wsem = (ws_0, ws_1, ws_2, ws_3, ws_4, ws_5)
    gsem = (gs_0, gs_1, gs_2)
    ssem = (ss_0, ss_1, ss_2)

    def start_edata(j, be):
        pltpu.async_copy(src_hbm.at[wid, j], src_v.at[be], esem[be])
        pltpu.async_copy(dst_hbm.at[wid, j], dst_v.at[be], dsem[be])
        if with_weight:
            pltpu.async_copy(w_hbm.at[wid, j], w_v.at[be], wsem[be])

    def wait_edata(j, be):
        pltpu.make_async_copy(src_hbm.at[wid, j], src_v.at[be],
                              esem[be]).wait()
        pltpu.make_async_copy(dst_hbm.at[wid, j], dst_v.at[be],
                              dsem[be]).wait()
        if with_weight:
            pltpu.make_async_copy(w_hbm.at[wid, j], w_v.at[be],
                                  wsem[be]).wait()

    def start_gather(j, b, be):
        pltpu.async_copy(y_hbm.at[src_v.at[be, 0]], rows[b], gsem[b])

    def wait_gather(j, b, be):
        pltpu.make_async_copy(y_hbm.at[src_v.at[be, 0]], rows[b],
                              gsem[b]).wait()

    def scale(b, be):
        r = rows[b]

        def group_body(g, _):
            wv = w_v[be, 0, pl.ds(g * LANES, LANES)]
            for el in range(LANES):
                e = g * LANES + el
                wb = jnp.full((LANES,), wv[el], jnp.float32)
                for k in range(D // LANES):
                    sl = pl.ds(k * LANES, LANES)
                    r[e, sl] = r[e, sl] * wb
            return 0
        lax.fori_loop(0, C // LANES, group_body, 0)

    def start_scatter(j, b, be):
        pltpu.async_copy(rows[b], acc_sh.at[dst_v.at[be, 0]], ssem[b],
                         add=True)

    def wait_scatter(j, b, be):
        pltpu.make_async_copy(rows[b], acc_sh.at[dst_v.at[be, 0]],
                              ssem[b]).wait()

    def lane(j, b, be, first=False, do_g=True, do_e=True):
        if do_g:
            wait_edata(j + 2, (be + 2) % NE)
        wait_gather(j, b, be)
        if not first:
            wait_scatter(j - 1, (b + 2) % NB, (be + 5) % NE)
        if do_g:
            start_gather(j + 2, (b + 2) % NB, (be + 2) % NE)
        if do_e:
            start_edata(j + 4, (be + 4) % NE)
        if with_weight:
            scale(b, be)
        start_scatter(j, b, be)

    for k in range(4):
        start_edata(k, k)

    def zero_body(i, _):
        rows_2[i // 8, pl.ds((i % 8) * LANES, LANES)] = jnp.zeros(
            (LANES,), jnp.float32)
        return 0
    lax.fori_loop(0, C * 8, zero_body, 0)
    for z in range(RPT // C):
        pltpu.async_copy(rows_2, acc_sh.at[pl.ds(sid * RPT + z * C, C)],
                         ss_0)
    for k in range(2):
        wait_edata(k, k)
        start_gather(k, k, k)
    for z in range(RPT // C):
        pltpu.make_async_copy(rows_2, acc_sh.at[pl.ds(sid * RPT + z * C, C)],
                              ss_0).wait()
    plsc.subcore_barrier()

    lane(0, 0, 0, first=True)
    lane(1, 1, 1)

    RS = 2
    RR = (NCH - RS - 4) // NE

    def ring_body(t, _):
        j0 = RS + NE * t
        for l in range(NE):
            lane(j0 + l, (RS + l) % NB, (RS + l) % NE)
        return 0
    lax.fori_loop(0, RR, ring_body, 0)

    for j in range(RS + NE * RR, NCH):
        lane(j, j % NB, j % NE, do_g=(j + 2 < NCH), do_e=(j + 4 < NCH))
    wait_scatter(NCH - 1, (NCH - 1) % NB, (NCH - 1) % NE)
    plsc.subcore_barrier()

    def rd(z, b):
        return (acc_sh.at[pl.ds(sid * RPT + z * C, C)], rows[b], gsem[b])

    def wr(z, b):
        return (rows[b], out_hbm.at[cid, pl.ds(sid * RPT + z * C, C)],
                ssem[b])

    pltpu.async_copy(*rd(0, 0))
    for z in range(RPT // C):
        b = z % 2
        pltpu.make_async_copy(*rd(z, b)).wait()
        if z > 0:
            pltpu.make_async_copy(*wr(z - 1, 1 - b)).wait()
        pltpu.async_copy(*wr(z, b))
        if z + 1 < RPT // C:
            pltpu.async_copy(*rd(z + 1, 1 - b))
    pltpu.make_async_copy(*wr(RPT // C - 1, (RPT // C - 1) % 2)).wait()


def _make_sc_scatter(with_weight):
    mesh = plsc.VectorSubcoreMesh(core_axis_name="c", subcore_axis_name="s",
                                  num_cores=NC, num_subcores=NS)
    scratch = (
        [pltpu.VMEM((NE, 1, C), jnp.int32)]
        + [pltpu.VMEM((NE, 1, C), jnp.int32)]
        + [pltpu.VMEM((NE, 1, C), jnp.float32)]
        + [pltpu.VMEM((C, D), jnp.float32) for _ in range(NB)]
        + [pltpu.VMEM_SHARED((NP, D), jnp.float32)]
        + [pltpu.SemaphoreType.DMA] * (3 * NE + 2 * NB)
    )
    return pl.kernel(
        functools.partial(_sc_scatter_body, with_weight),
        out_type=jax.ShapeDtypeStruct((NC, NP, D), jnp.float32),
        mesh=mesh,
        scratch_types=scratch,
        name="sc_scatter_w" if with_weight else "sc_scatter",
    )


_sc_scatter_weighted = _make_sc_scatter(True)
_sc_scatter_plain = _make_sc_scatter(False)


def _mm(a, w):
    return lax.dot_general(a, w, (((1,), (1,)), ((), ())),
                           preferred_element_type=jnp.float32)


def _tc_root(x_ref, wq_ref, br_ref, bq_ref, r_ref):
    r_ref[...] = _mm(x_ref[...], wq_ref[...]) + br_ref[...] + bq_ref[...]


def _tc_merge(p_ref, r_ref, wr_ref, h_ref):
    agg = p_ref[0, :N, :] + p_ref[1, :N, :]
    h_ref[...] = jnp.maximum(_mm(agg, wr_ref[...]) + r_ref[...], 0.0)


_nd = jax.ShapeDtypeStruct((N, D), jnp.float32)
_tc_root_call = pl.pallas_call(_tc_root, out_shape=_nd)
_tc_merge_call = pl.pallas_call(_tc_merge, out_shape=_nd)


def kernel(x, edge_index, edge_weight, W1_rel, b1_rel, W1_root, b1_root,
           W2_rel, b2_rel, W2_root, b2_root):
    src = edge_index[0].reshape(NW, NCH, 1, C)
    dst = edge_index[1].reshape(NW, NCH, 1, C)
    w = edge_weight.reshape(NW, NCH, 1, C)

    p1 = _sc_scatter_weighted(x, src, dst, w)
    r1 = _tc_root_call(x, W1_root, b1_rel.reshape(1, D), b1_root.reshape(1, D))
    h1 = _tc_merge_call(p1, r1, W1_rel)
    p2 = _sc_scatter_plain(h1, src, dst, w)
    r2 = _tc_root_call(h1, W2_root, b2_rel.reshape(1, D),
                       b2_root.reshape(1, D))
    return _tc_merge_call(p2, r2, W2_rel)
